# Initial kernel scaffold; baseline (speedup 1.0000x reference)
import jax
import jax.numpy as jnp
from jax.experimental import pallas as pl


def kernel(features0, features1, W0, b0, W1, b1, Wc0, bc0, Wc1, bc1, Wa, ba, qvec, Wfc, bfc, deg1, deg2, edge_index0, edge_index1, target_idx):
    raise NotImplementedError("write your pallas kernel here")


# speedup vs baseline: 2.4139x; 2.4139x over previous
import jax
import jax.numpy as jnp
from jax import lax
from jax.experimental import pallas as pl
from jax.experimental.pallas import tpu as pltpu
from jax.experimental.pallas import tpu_sc as plsc

N = 50000
N0 = 25000
E = 800000
D_FEAT = 128
HID = 32
NDEG = 64
OUT = 64
T = 8192

NPAD = 50176
NG = NPAD // 4
NSUB = 16
NCORE = 2
E_PER_SUB = E // NSUB
CHUNK = 80
N_CHUNKS = E_PER_SUB // CHUNK
ACC_STRIPE = NG // NSUB
CNT_STRIPE = NPAD // NSUB
CNT_PIECE = 784
T_PER_W = T // (NCORE * NSUB)


def _leaky(x):
    return jnp.where(x > 0, x, 0.7 * x)


BLK_A = 1000


def _table_body(feat, d1, d2, wst, bst, wcst, bcst, out):
    j = pl.program_id(0)
    c = pl.program_id(1)
    blk = feat.shape[0]
    x = (jnp.dot(feat[...], wst[...][0], preferred_element_type=jnp.float32)
         + bst[...][0])
    io = lax.broadcasted_iota(jnp.int32, (blk, NDEG), 1)
    oh1 = (d1[...] == io).astype(jnp.float32)
    oh2 = (d2[...] == io).astype(jnp.float32)
    e = (jnp.dot(oh1, wcst[...][0][:NDEG], preferred_element_type=jnp.float32)
         + jnp.dot(oh2, wcst[...][0][NDEG:], preferred_element_type=jnp.float32)
         + bcst[...][0])
    val = jnp.where(c == 0, _leaky(x), _leaky(e))
    out[...] = jnp.zeros((blk, 4 * HID), jnp.float32)
    for jj in range(4):
        @pl.when(j == jj)
        def _():
            out[:, jj * HID:(jj + 1) * HID] = val


def _build_table(feat, deg1c, deg2c, wst, bst, wcst, bcst):
    nblk = N // BLK_A
    tblk = N0 // BLK_A
    return pl.pallas_call(
        _table_body,
        grid=(4, 2, nblk),
        in_specs=[
            pl.BlockSpec((BLK_A, D_FEAT), lambda j, c, i: (i, 0)),
            pl.BlockSpec((BLK_A, 1), lambda j, c, i: (i, 0)),
            pl.BlockSpec((BLK_A, 1), lambda j, c, i: (i, 0)),
            pl.BlockSpec((1, D_FEAT, HID), lambda j, c, i: (i // tblk, 0, 0)),
            pl.BlockSpec((1, 1, HID), lambda j, c, i: (i // tblk, 0, 0)),
            pl.BlockSpec((1, 2 * NDEG, HID), lambda j, c, i: (i // tblk, 0, 0)),
            pl.BlockSpec((1, 1, HID), lambda j, c, i: (i // tblk, 0, 0)),
        ],
        out_specs=pl.BlockSpec((BLK_A, 4 * HID),
                               lambda j, c, i: (j * 2 * nblk + c * nblk + i, 0)),
        out_shape=jax.ShapeDtypeStruct((8 * N, 4 * HID), jnp.float32),
    )(feat, deg1c, deg2c, wst, bst, wcst, bcst)


def _fill_f32_1d(ref, nwords, value):
    def body(i, _):
        ref[pl.ds(i * 16, 16)] = jnp.full((16,), value, jnp.float32)
        return 0
    lax.fori_loop(0, nwords // 16, body, 0)


def _edge_kernel_body(btable, src0, dst0, src1, dst1, zr,
                      s00, s01, s10, s11, cnt0, cnt1,
                      idxs, idxd, gidx, sidx, rows, ones1, cbuf,
                      acc, cacc, sem):
    cid = lax.axis_index("c")
    sid = lax.axis_index("s")
    tbase = cid * N
    b = sid * ACC_STRIPE
    cb = sid * CNT_STRIPE

    _fill_f32_1d(ones1, CHUNK, 1.0)

    def one_pass(src, dst, s_out, c_out, with_cnt):
        pltpu.sync_copy(zr, acc.at[pl.ds(b, ACC_STRIPE)])
        if with_cnt:
            _fill_f32_1d(cbuf, CNT_PIECE, 0.0)
            for k in range(CNT_STRIPE // CNT_PIECE):
                pltpu.sync_copy(cbuf, cacc.at[pl.ds(cb + k * CNT_PIECE, CNT_PIECE)])
        plsc.subcore_barrier()

        def body(j, _):
            off = sid * E_PER_SUB + j * CHUNK
            pltpu.sync_copy(src.at[pl.ds(off, CHUNK)], idxs)
            pltpu.sync_copy(dst.at[pl.ds(off, CHUNK)], idxd)

            def vbody(k, _):
                s = idxs[pl.ds(k * 16, 16)]
                d = idxd[pl.ds(k * 16, 16)]
                gidx[pl.ds(k * 16, 16)] = s + tbase + (d & 3) * (2 * N)
                sidx[pl.ds(k * 16, 16)] = lax.shift_right_logical(d, 2)
                return 0
            lax.fori_loop(0, CHUNK // 16, vbody, 0)

            pltpu.async_copy(btable.at[gidx], rows, sem).wait()
            pltpu.sync_copy(rows, acc.at[sidx], add=True)
            if with_cnt:
                pltpu.sync_copy(ones1, cacc.at[idxd], add=True)
            return 0
        lax.fori_loop(0, N_CHUNKS, body, 0)
        plsc.subcore_barrier()

        pltpu.sync_copy(acc.at[pl.ds(b, ACC_STRIPE)], s_out.at[pl.ds(b, ACC_STRIPE)])
        if with_cnt:
            for k in range(CNT_STRIPE // CNT_PIECE):
                pltpu.sync_copy(cacc.at[pl.ds(cb + k * CNT_PIECE, CNT_PIECE)], cbuf)
                pltpu.sync_copy(cbuf, c_out.at[pl.ds(cb + k * CNT_PIECE, CNT_PIECE)])

    @pl.when(cid == 0)
    def _():
        one_pass(src0, dst0, s00, cnt0, True)
        one_pass(src1, dst1, s10, cnt0, False)

    @pl.when(cid == 1)
    def _():
        one_pass(src0, dst0, s01, cnt1, False)
        one_pass(src1, dst1, s11, cnt1, True)


def _edge_aggregate(btable, src0, dst0, src1, dst1, zr):
    mesh = plsc.VectorSubcoreMesh(core_axis_name="c", subcore_axis_name="s")
    f = pl.kernel(
        _edge_kernel_body,
        out_type=[
            jax.ShapeDtypeStruct((NG, 4 * HID), jnp.float32),
            jax.ShapeDtypeStruct((NG, 4 * HID), jnp.float32),
            jax.ShapeDtypeStruct((NG, 4 * HID), jnp.float32),
            jax.ShapeDtypeStruct((NG, 4 * HID), jnp.float32),
            jax.ShapeDtypeStruct((NPAD,), jnp.float32),
            jax.ShapeDtypeStruct((NPAD,), jnp.float32),
        ],
        mesh=mesh,
        scratch_types=[
            pltpu.VMEM((CHUNK,), jnp.int32),
            pltpu.VMEM((CHUNK,), jnp.int32),
            pltpu.VMEM((CHUNK,), jnp.int32),
            pltpu.VMEM((CHUNK,), jnp.int32),
            pltpu.VMEM((CHUNK, 4 * HID), jnp.float32),
            pltpu.VMEM((CHUNK,), jnp.float32),
            pltpu.VMEM((CNT_PIECE,), jnp.float32),
            pltpu.VMEM_SHARED((NG, 4 * HID), jnp.float32),
            pltpu.VMEM_SHARED((NPAD,), jnp.float32),
            pltpu.SemaphoreType.DMA,
        ],
    )
    return f(btable, src0, dst0, src1, dst1, zr)


def _attn_body(s00, s01, s10, s11, c0, c1, wa, ba, qv, hcat, o0, o1):
    i = pl.program_id(0)
    inv0 = 1.0 / jnp.maximum(c0[...], 1.0)
    inv1 = 1.0 / jnp.maximum(c1[...], 1.0)
    h0 = jnp.concatenate([s00[...], s01[...]], axis=1) * inv0
    h1 = jnp.concatenate([s10[...], s11[...]], axis=1) * inv1
    hcat[...] = jnp.concatenate([h0, h1], axis=1)
    t0 = jnp.tanh(jnp.dot(h0, wa[...], preferred_element_type=jnp.float32) + ba[...])
    t1 = jnp.tanh(jnp.dot(h1, wa[...], preferred_element_type=jnp.float32) + ba[...])
    p0 = jnp.sum(jnp.dot(t0, qv[...], preferred_element_type=jnp.float32))
    p1 = jnp.sum(jnp.dot(t1, qv[...], preferred_element_type=jnp.float32))

    @pl.when(i == 0)
    def _():
        o0[...] = jnp.zeros_like(o0)
        o1[...] = jnp.zeros_like(o1)

    o0[...] += p0
    o1[...] += p1


def _attn_scores(s00, s01, s10, s11, c0col, c1col, wa, ba2, qv):
    blk = 1000
    grid = N // blk
    return pl.pallas_call(
        _attn_body,
        grid=(grid,),
        in_specs=[
            pl.BlockSpec((blk, HID), lambda i: (i, 0)),
            pl.BlockSpec((blk, HID), lambda i: (i, 0)),
            pl.BlockSpec((blk, HID), lambda i: (i, 0)),
            pl.BlockSpec((blk, HID), lambda i: (i, 0)),
            pl.BlockSpec((blk, 1), lambda i: (i, 0)),
            pl.BlockSpec((blk, 1), lambda i: (i, 0)),
            pl.BlockSpec((2 * HID, 2 * HID), lambda i: (0, 0)),
            pl.BlockSpec((1, 2 * HID), lambda i: (0, 0)),
            pl.BlockSpec((2 * HID, 1), lambda i: (0, 0)),
        ],
        out_specs=[
            pl.BlockSpec((blk, 4 * HID), lambda i: (i, 0)),
            pl.BlockSpec((1, 1), lambda i: (0, 0)),
            pl.BlockSpec((1, 1), lambda i: (0, 0)),
        ],
        out_shape=[
            jax.ShapeDtypeStruct((N, 4 * HID), jnp.float32),
            jax.ShapeDtypeStruct((1, 1), jnp.float32),
            jax.ShapeDtypeStruct((1, 1), jnp.float32),
        ],
    )(s00, s01, s10, s11, c0col, c1col, wa, ba2, qv)


def _gather_body(hcat, tgt, g, tix, buf, sem):
    cid = lax.axis_index("c")
    sid = lax.axis_index("s")
    wid = sid * NCORE + cid
    base = wid * T_PER_W
    pltpu.sync_copy(tgt.at[pl.ds(base, T_PER_W)], tix)
    pltpu.async_copy(hcat.at[tix], buf, sem).wait()
    pltpu.sync_copy(buf, g.at[pl.ds(base, T_PER_W)])


def _gather_targets(hcat, tgt):
    mesh = plsc.VectorSubcoreMesh(core_axis_name="c", subcore_axis_name="s")
    f = pl.kernel(
        _gather_body,
        out_type=jax.ShapeDtypeStruct((T, 4 * HID), jnp.float32),
        mesh=mesh,
        scratch_types=[
            pltpu.VMEM((T_PER_W,), jnp.int32),
            pltpu.VMEM((T_PER_W, 4 * HID), jnp.float32),
            pltpu.SemaphoreType.DMA,
        ],
    )
    return f(hcat, tgt)


def _final_body(g, s0, s1, wfc, bfc, lo, ho):
    sv0 = s0[0, 0] / N
    sv1 = s1[0, 0] / N
    m = jnp.maximum(sv0, sv1)
    e0 = jnp.exp(sv0 - m)
    e1 = jnp.exp(sv1 - m)
    b0 = e0 / (e0 + e1)
    b1 = e1 / (e0 + e1)
    h = b0 * g[...][:, : 2 * HID] + b1 * g[...][:, 2 * HID:]
    ho[...] = h
    lo[...] = jnp.dot(h, wfc[...], preferred_element_type=jnp.float32) + bfc[...]


def _final(g, s0, s1, wfc, bfc2):
    blk = 2048
    grid = T // blk
    return pl.pallas_call(
        _final_body,
        grid=(grid,),
        in_specs=[
            pl.BlockSpec((blk, 4 * HID), lambda i: (i, 0)),
            pl.BlockSpec((1, 1), lambda i: (0, 0)),
            pl.BlockSpec((1, 1), lambda i: (0, 0)),
            pl.BlockSpec((2 * HID, OUT), lambda i: (0, 0)),
            pl.BlockSpec((1, OUT), lambda i: (0, 0)),
        ],
        out_specs=[
            pl.BlockSpec((blk, OUT), lambda i: (i, 0)),
            pl.BlockSpec((blk, 2 * HID), lambda i: (i, 0)),
        ],
        out_shape=[
            jax.ShapeDtypeStruct((T, OUT), jnp.float32),
            jax.ShapeDtypeStruct((T, 2 * HID), jnp.float32),
        ],
    )(g, s0, s1, wfc, bfc2)


def kernel(features0, features1, W0, b0, W1, b1, Wc0, bc0, Wc1, bc1,
           Wa, ba, qvec, Wfc, bfc, deg1, deg2, edge_index0, edge_index1,
           target_idx):
    d1c = deg1.reshape(-1, 1).astype(jnp.int32)
    d2c = deg2.reshape(-1, 1).astype(jnp.int32)
    feat = jnp.concatenate([features0, features1], axis=0)
    wst = jnp.stack([W0, W1])
    bst = jnp.stack([b0.reshape(1, -1), b1.reshape(1, -1)])
    wcst = jnp.stack([Wc0, Wc1])
    bcst = jnp.stack([bc0.reshape(1, -1), bc1.reshape(1, -1)])

    btable = _build_table(feat, d1c, d2c, wst, bst, wcst, bcst)

    src0 = edge_index0[0].astype(jnp.int32)
    dst0 = edge_index0[1].astype(jnp.int32)
    src1 = edge_index1[0].astype(jnp.int32)
    dst1 = edge_index1[1].astype(jnp.int32)

    zr = jnp.zeros((ACC_STRIPE, 4 * HID), jnp.float32)
    a00, a01, a10, a11, cnt0, cnt1 = _edge_aggregate(
        btable, src0, dst0, src1, dst1, zr)

    s00 = a00.reshape(NPAD, HID)
    s01 = a01.reshape(NPAD, HID)
    s10 = a10.reshape(NPAD, HID)
    s11 = a11.reshape(NPAD, HID)

    c0col = cnt0[:N].reshape(-1, 1)
    c1col = cnt1[:N].reshape(-1, 1)
    hcat, o0, o1 = _attn_scores(s00[:N], s01[:N], s10[:N], s11[:N],
                                c0col, c1col,
                                Wa, ba.reshape(1, -1), qvec.reshape(-1, 1))

    g = _gather_targets(hcat, target_idx.astype(jnp.int32))

    logits, h = _final(g, o0, o1, Wfc, bfc.reshape(1, -1))
    return logits, h


# speedup vs baseline: 3.6086x; 1.4949x over previous
import jax
import jax.numpy as jnp
from jax import lax
from jax.experimental import pallas as pl
from jax.experimental.pallas import tpu as pltpu
from jax.experimental.pallas import tpu_sc as plsc

N = 50000
N0 = 25000
E = 800000
D_FEAT = 128
HID = 32
NDEG = 64
OUT = 64
T = 8192

NPAD = 50176
NG = NPAD // 4
NSUB = 16
NCORE = 2
E_PER_SUB = E // NSUB
CHUNK = 80
N_CHUNKS = E_PER_SUB // CHUNK
ACC_STRIPE = NG // NSUB
CNT_STRIPE = NPAD // NSUB
CNT_PIECE = 784
T_PER_W = T // (NCORE * NSUB)


def _leaky(x):
    return jnp.where(x > 0, x, 0.7 * x)


BLK_A = 1000


def _table_body(feat, d1, d2, wst, bst, wcst, bcst, out):
    j = pl.program_id(0)
    c = pl.program_id(1)
    blk = feat.shape[0]
    x = (jnp.dot(feat[...], wst[...][0], preferred_element_type=jnp.float32)
         + bst[...][0])
    io = lax.broadcasted_iota(jnp.int32, (blk, NDEG), 1)
    oh1 = (d1[...] == io).astype(jnp.float32)
    oh2 = (d2[...] == io).astype(jnp.float32)
    e = (jnp.dot(oh1, wcst[...][0][:NDEG], preferred_element_type=jnp.float32)
         + jnp.dot(oh2, wcst[...][0][NDEG:], preferred_element_type=jnp.float32)
         + bcst[...][0])
    val = jnp.where(c == 0, _leaky(x), _leaky(e))
    out[...] = jnp.zeros((blk, 4 * HID), jnp.float32)
    for jj in range(4):
        @pl.when(j == jj)
        def _():
            out[:, jj * HID:(jj + 1) * HID] = val


def _build_table(feat, deg1c, deg2c, wst, bst, wcst, bcst):
    nblk = N // BLK_A
    tblk = N0 // BLK_A
    return pl.pallas_call(
        _table_body,
        grid=(4, 2, nblk),
        in_specs=[
            pl.BlockSpec((BLK_A, D_FEAT), lambda j, c, i: (i, 0)),
            pl.BlockSpec((BLK_A, 1), lambda j, c, i: (i, 0)),
            pl.BlockSpec((BLK_A, 1), lambda j, c, i: (i, 0)),
            pl.BlockSpec((1, D_FEAT, HID), lambda j, c, i: (i // tblk, 0, 0)),
            pl.BlockSpec((1, 1, HID), lambda j, c, i: (i // tblk, 0, 0)),
            pl.BlockSpec((1, 2 * NDEG, HID), lambda j, c, i: (i // tblk, 0, 0)),
            pl.BlockSpec((1, 1, HID), lambda j, c, i: (i // tblk, 0, 0)),
        ],
        out_specs=pl.BlockSpec((BLK_A, 4 * HID),
                               lambda j, c, i: (j * 2 * nblk + c * nblk + i, 0)),
        out_shape=jax.ShapeDtypeStruct((8 * N, 4 * HID), jnp.float32),
    )(feat, deg1c, deg2c, wst, bst, wcst, bcst)


def _fill_f32_1d(ref, nwords, value):
    def body(i, _):
        ref[pl.ds(i * 16, 16)] = jnp.full((16,), value, jnp.float32)
        return 0
    lax.fori_loop(0, nwords // 16, body, 0)


def _edge_kernel_body(btable, src0, dst0, src1, dst1, zr,
                      s00, s01, s10, s11, cnt0, cnt1,
                      idxs0, idxd0, gidx0, sidx0, rows0,
                      idxs1, idxd1, gidx1, sidx1, rows1,
                      ones1, cbuf, acc, cacc,
                      si0, sd0, sg0, ss0, sc0, si1, sd1, sg1, ss1, sc1):
    cid = lax.axis_index("c")
    sid = lax.axis_index("s")
    tbase = cid * N
    b = sid * ACC_STRIPE
    cb = sid * CNT_STRIPE

    idxs = (idxs0, idxs1)
    idxd = (idxd0, idxd1)
    gidx = (gidx0, gidx1)
    sidx = (sidx0, sidx1)
    rows = (rows0, rows1)
    si = (si0, si1)
    sd = (sd0, sd1)
    sg = (sg0, sg1)
    ss = (ss0, ss1)
    sc = (sc0, sc1)

    _fill_f32_1d(ones1, CHUNK, 1.0)

    def compute_idx(q):
        def vbody(k, _):
            s = idxs[q][pl.ds(k * 16, 16)]
            d = idxd[q][pl.ds(k * 16, 16)]
            gidx[q][pl.ds(k * 16, 16)] = s + tbase + (d & 3) * (2 * N)
            sidx[q][pl.ds(k * 16, 16)] = lax.shift_right_logical(d, 2)
            return 0
        lax.fori_loop(0, CHUNK // 16, vbody, 0)

    def one_pass(src, dst, s_out, c_out, with_cnt):
        pltpu.sync_copy(zr, acc.at[pl.ds(b, ACC_STRIPE)])
        if with_cnt:
            _fill_f32_1d(cbuf, CNT_PIECE, 0.0)
            for k in range(CNT_STRIPE // CNT_PIECE):
                pltpu.sync_copy(cbuf, cacc.at[pl.ds(cb + k * CNT_PIECE, CNT_PIECE)])
        plsc.subcore_barrier()

        def body(jp, _):
            off = sid * E_PER_SUB + jp * (2 * CHUNK)
            hi = []
            for q in (0, 1):
                hi.append((
                    pltpu.async_copy(src.at[pl.ds(off + q * CHUNK, CHUNK)],
                                     idxs[q], si[q]),
                    pltpu.async_copy(dst.at[pl.ds(off + q * CHUNK, CHUNK)],
                                     idxd[q], sd[q])))
            hg = [None, None]
            for q in (0, 1):
                hi[q][0].wait()
                hi[q][1].wait()
                compute_idx(q)
                hg[q] = pltpu.async_copy(btable.at[gidx[q]], rows[q], sg[q])
            hs = [None, None]
            hc = [None, None]
            for q in (0, 1):
                hg[q].wait()
                hs[q] = pltpu.async_copy(rows[q], acc.at[sidx[q]], ss[q],
                                         add=True)
                if with_cnt:
                    hc[q] = pltpu.async_copy(ones1, cacc.at[idxd[q]], sc[q],
                                             add=True)
            for q in (0, 1):
                hs[q].wait()
                if with_cnt:
                    hc[q].wait()
            return 0
        lax.fori_loop(0, N_CHUNKS // 2, body, 0)

        if N_CHUNKS % 2:
            off = sid * E_PER_SUB + (N_CHUNKS - 1) * CHUNK
            pltpu.sync_copy(src.at[pl.ds(off, CHUNK)], idxs[0])
            pltpu.sync_copy(dst.at[pl.ds(off, CHUNK)], idxd[0])
            compute_idx(0)
            pltpu.async_copy(btable.at[gidx[0]], rows[0], sg[0]).wait()
            pltpu.sync_copy(rows[0], acc.at[sidx[0]], add=True)
            if with_cnt:
                pltpu.sync_copy(ones1, cacc.at[idxd[0]], add=True)
        plsc.subcore_barrier()

        pltpu.sync_copy(acc.at[pl.ds(b, ACC_STRIPE)], s_out.at[pl.ds(b, ACC_STRIPE)])
        if with_cnt:
            for k in range(CNT_STRIPE // CNT_PIECE):
                pltpu.sync_copy(cacc.at[pl.ds(cb + k * CNT_PIECE, CNT_PIECE)], cbuf)
                pltpu.sync_copy(cbuf, c_out.at[pl.ds(cb + k * CNT_PIECE, CNT_PIECE)])

    @pl.when(cid == 0)
    def _():
        one_pass(src0, dst0, s00, cnt0, True)
        one_pass(src1, dst1, s10, cnt0, False)

    @pl.when(cid == 1)
    def _():
        one_pass(src0, dst0, s01, cnt1, False)
        one_pass(src1, dst1, s11, cnt1, True)


def _edge_aggregate(btable, src0, dst0, src1, dst1, zr):
    mesh = plsc.VectorSubcoreMesh(core_axis_name="c", subcore_axis_name="s")
    f = pl.kernel(
        _edge_kernel_body,
        out_type=[
            jax.ShapeDtypeStruct((NG, 4 * HID), jnp.float32),
            jax.ShapeDtypeStruct((NG, 4 * HID), jnp.float32),
            jax.ShapeDtypeStruct((NG, 4 * HID), jnp.float32),
            jax.ShapeDtypeStruct((NG, 4 * HID), jnp.float32),
            jax.ShapeDtypeStruct((NPAD,), jnp.float32),
            jax.ShapeDtypeStruct((NPAD,), jnp.float32),
        ],
        mesh=mesh,
        scratch_types=(
            [pltpu.VMEM((CHUNK,), jnp.int32)] * 4
            + [pltpu.VMEM((CHUNK, 4 * HID), jnp.float32)]
            + [pltpu.VMEM((CHUNK,), jnp.int32)] * 4
            + [pltpu.VMEM((CHUNK, 4 * HID), jnp.float32)]
            + [pltpu.VMEM((CHUNK,), jnp.float32),
               pltpu.VMEM((CNT_PIECE,), jnp.float32),
               pltpu.VMEM_SHARED((NG, 4 * HID), jnp.float32),
               pltpu.VMEM_SHARED((NPAD,), jnp.float32)]
            + [pltpu.SemaphoreType.DMA] * 10
        ),
    )
    return f(btable, src0, dst0, src1, dst1, zr)


def _attn_body(s00, s01, s10, s11, c0, c1, wa, ba, qv, hcat, o0, o1):
    i = pl.program_id(0)
    inv0 = 1.0 / jnp.maximum(c0[...], 1.0)
    inv1 = 1.0 / jnp.maximum(c1[...], 1.0)
    h0 = jnp.concatenate([s00[...], s01[...]], axis=1) * inv0
    h1 = jnp.concatenate([s10[...], s11[...]], axis=1) * inv1
    hcat[...] = jnp.concatenate([h0, h1], axis=1)
    t0 = jnp.tanh(jnp.dot(h0, wa[...], preferred_element_type=jnp.float32) + ba[...])
    t1 = jnp.tanh(jnp.dot(h1, wa[...], preferred_element_type=jnp.float32) + ba[...])
    p0 = jnp.sum(jnp.dot(t0, qv[...], preferred_element_type=jnp.float32))
    p1 = jnp.sum(jnp.dot(t1, qv[...], preferred_element_type=jnp.float32))

    @pl.when(i == 0)
    def _():
        o0[...] = jnp.zeros_like(o0)
        o1[...] = jnp.zeros_like(o1)

    o0[...] += p0
    o1[...] += p1


def _attn_scores(s00, s01, s10, s11, c0col, c1col, wa, ba2, qv):
    blk = 1000
    grid = N // blk
    return pl.pallas_call(
        _attn_body,
        grid=(grid,),
        in_specs=[
            pl.BlockSpec((blk, HID), lambda i: (i, 0)),
            pl.BlockSpec((blk, HID), lambda i: (i, 0)),
            pl.BlockSpec((blk, HID), lambda i: (i, 0)),
            pl.BlockSpec((blk, HID), lambda i: (i, 0)),
            pl.BlockSpec((blk, 1), lambda i: (i, 0)),
            pl.BlockSpec((blk, 1), lambda i: (i, 0)),
            pl.BlockSpec((2 * HID, 2 * HID), lambda i: (0, 0)),
            pl.BlockSpec((1, 2 * HID), lambda i: (0, 0)),
            pl.BlockSpec((2 * HID, 1), lambda i: (0, 0)),
        ],
        out_specs=[
            pl.BlockSpec((blk, 4 * HID), lambda i: (i, 0)),
            pl.BlockSpec((1, 1), lambda i: (0, 0)),
            pl.BlockSpec((1, 1), lambda i: (0, 0)),
        ],
        out_shape=[
            jax.ShapeDtypeStruct((N, 4 * HID), jnp.float32),
            jax.ShapeDtypeStruct((1, 1), jnp.float32),
            jax.ShapeDtypeStruct((1, 1), jnp.float32),
        ],
    )(s00, s01, s10, s11, c0col, c1col, wa, ba2, qv)


def _gather_body(hcat, tgt, g, tix, buf, sem):
    cid = lax.axis_index("c")
    sid = lax.axis_index("s")
    wid = sid * NCORE + cid
    base = wid * T_PER_W
    pltpu.sync_copy(tgt.at[pl.ds(base, T_PER_W)], tix)
    pltpu.async_copy(hcat.at[tix], buf, sem).wait()
    pltpu.sync_copy(buf, g.at[pl.ds(base, T_PER_W)])


def _gather_targets(hcat, tgt):
    mesh = plsc.VectorSubcoreMesh(core_axis_name="c", subcore_axis_name="s")
    f = pl.kernel(
        _gather_body,
        out_type=jax.ShapeDtypeStruct((T, 4 * HID), jnp.float32),
        mesh=mesh,
        scratch_types=[
            pltpu.VMEM((T_PER_W,), jnp.int32),
            pltpu.VMEM((T_PER_W, 4 * HID), jnp.float32),
            pltpu.SemaphoreType.DMA,
        ],
    )
    return f(hcat, tgt)


def _final_body(g, s0, s1, wfc, bfc, lo, ho):
    sv0 = s0[0, 0] / N
    sv1 = s1[0, 0] / N
    m = jnp.maximum(sv0, sv1)
    e0 = jnp.exp(sv0 - m)
    e1 = jnp.exp(sv1 - m)
    b0 = e0 / (e0 + e1)
    b1 = e1 / (e0 + e1)
    h = b0 * g[...][:, : 2 * HID] + b1 * g[...][:, 2 * HID:]
    ho[...] = h
    lo[...] = jnp.dot(h, wfc[...], preferred_element_type=jnp.float32) + bfc[...]


def _final(g, s0, s1, wfc, bfc2):
    blk = 2048
    grid = T // blk
    return pl.pallas_call(
        _final_body,
        grid=(grid,),
        in_specs=[
            pl.BlockSpec((blk, 4 * HID), lambda i: (i, 0)),
            pl.BlockSpec((1, 1), lambda i: (0, 0)),
            pl.BlockSpec((1, 1), lambda i: (0, 0)),
            pl.BlockSpec((2 * HID, OUT), lambda i: (0, 0)),
            pl.BlockSpec((1, OUT), lambda i: (0, 0)),
        ],
        out_specs=[
            pl.BlockSpec((blk, OUT), lambda i: (i, 0)),
            pl.BlockSpec((blk, 2 * HID), lambda i: (i, 0)),
        ],
        out_shape=[
            jax.ShapeDtypeStruct((T, OUT), jnp.float32),
            jax.ShapeDtypeStruct((T, 2 * HID), jnp.float32),
        ],
    )(g, s0, s1, wfc, bfc2)


def kernel(features0, features1, W0, b0, W1, b1, Wc0, bc0, Wc1, bc1,
           Wa, ba, qvec, Wfc, bfc, deg1, deg2, edge_index0, edge_index1,
           target_idx):
    d1c = deg1.reshape(-1, 1).astype(jnp.int32)
    d2c = deg2.reshape(-1, 1).astype(jnp.int32)
    feat = jnp.concatenate([features0, features1], axis=0)
    wst = jnp.stack([W0, W1])
    bst = jnp.stack([b0.reshape(1, -1), b1.reshape(1, -1)])
    wcst = jnp.stack([Wc0, Wc1])
    bcst = jnp.stack([bc0.reshape(1, -1), bc1.reshape(1, -1)])

    btable = _build_table(feat, d1c, d2c, wst, bst, wcst, bcst)

    src0 = edge_index0[0].astype(jnp.int32)
    dst0 = edge_index0[1].astype(jnp.int32)
    src1 = edge_index1[0].astype(jnp.int32)
    dst1 = edge_index1[1].astype(jnp.int32)

    zr = jnp.zeros((ACC_STRIPE, 4 * HID), jnp.float32)
    a00, a01, a10, a11, cnt0, cnt1 = _edge_aggregate(
        btable, src0, dst0, src1, dst1, zr)

    s00 = a00.reshape(NPAD, HID)
    s01 = a01.reshape(NPAD, HID)
    s10 = a10.reshape(NPAD, HID)
    s11 = a11.reshape(NPAD, HID)

    c0col = cnt0[:N].reshape(-1, 1)
    c1col = cnt1[:N].reshape(-1, 1)
    hcat, o0, o1 = _attn_scores(s00[:N], s01[:N], s10[:N], s11[:N],
                                c0col, c1col,
                                Wa, ba.reshape(1, -1), qvec.reshape(-1, 1))

    g = _gather_targets(hcat, target_idx.astype(jnp.int32))

    logits, h = _final(g, o0, o1, Wfc, bfc.reshape(1, -1))
    return logits, h
